# baseline (speedup 1.0000x reference)
import functools

import jax
import jax.numpy as jnp
from jax import lax
from jax.experimental import pallas as pl
from jax.experimental.pallas import tpu as pltpu

VOCAB = 1000000
NUM_SPEAKERS = 100000
EMBED_DIM = 64
CONTEXT = 200
HIDDEN = 128
NROWS = CONTEXT + 1
IN1 = NROWS * EMBED_DIM
NSEM = 8


def _fused_body(widx_ref, spk_ref, wtab_ref, stab_ref, w1_ref, b1_ref,
                w2_ref, b2_ref, out_ref, rows_v, sem):
    pltpu.make_async_copy(
        stab_ref.at[pl.ds(spk_ref[0], 1)], rows_v.at[pl.ds(0, 1)], sem.at[0]
    ).start()

    for k in range(NSEM):
        def fire(j, carry, k=k):
            i = j * NSEM + k
            pltpu.make_async_copy(
                wtab_ref.at[pl.ds(widx_ref[i], 1)],
                rows_v.at[pl.ds(i + 1, 1)],
                sem.at[k],
            ).start()
            return carry

        lax.fori_loop(0, CONTEXT // NSEM, fire, 0)

    for k in range(NSEM):
        pltpu.make_async_copy(
            wtab_ref.at[pl.ds(0, CONTEXT // NSEM)],
            rows_v.at[pl.ds(1, CONTEXT // NSEM)],
            sem.at[k],
        ).wait()
    pltpu.make_async_copy(
        wtab_ref.at[pl.ds(0, 1)], rows_v.at[pl.ds(0, 1)], sem.at[0]
    ).wait()

    def accum(c, h):
        x_c = rows_v[pl.ds(c, 1)]
        w_c = w1_ref[pl.ds(c, 1)][0]
        return h + jnp.dot(x_c, w_c, preferred_element_type=jnp.float32)

    h = lax.fori_loop(0, NROWS, accum, jnp.zeros((1, HIDDEN), jnp.float32))
    h = jnp.maximum(h + b1_ref[...], 0.0)
    o = jnp.dot(h, w2_ref[...], preferred_element_type=jnp.float32)
    out_ref[...] = jax.nn.sigmoid(o + b2_ref[...])


@functools.partial(jax.jit, static_argnames=())
def kernel(speaker_code, word_indices, word_table, speaker_table, W1, b1, W2, b2):
    grid_spec = pltpu.PrefetchScalarGridSpec(
        num_scalar_prefetch=2,
        grid=(1,),
        in_specs=[
            pl.BlockSpec(memory_space=pl.ANY),
            pl.BlockSpec(memory_space=pl.ANY),
            pl.BlockSpec((NROWS, EMBED_DIM, HIDDEN), lambda i, *_: (0, 0, 0)),
            pl.BlockSpec((1, HIDDEN), lambda i, *_: (0, 0)),
            pl.BlockSpec((HIDDEN, 1), lambda i, *_: (0, 0)),
            pl.BlockSpec((1, 1), lambda i, *_: (0, 0)),
        ],
        out_specs=pl.BlockSpec((1, 1), lambda i, *_: (0, 0)),
        scratch_shapes=[
            pltpu.VMEM((NROWS, EMBED_DIM), jnp.float32),
            pltpu.SemaphoreType.DMA((NSEM,)),
        ],
    )
    return pl.pallas_call(
        _fused_body,
        grid_spec=grid_spec,
        out_shape=jax.ShapeDtypeStruct((1, 1), jnp.float32),
    )(word_indices.astype(jnp.int32), speaker_code.astype(jnp.int32),
      word_table, speaker_table, W1.reshape(NROWS, EMBED_DIM, HIDDEN),
      b1.reshape(1, HIDDEN), W2, b2.reshape(1, 1))
